# baseline (speedup 1.0000x reference)
import functools

import numpy as np
import jax
import jax.numpy as jnp
from jax import lax
from jax.experimental import pallas as pl
from jax.experimental.pallas import tpu as pltpu
from jax.experimental.pallas import tpu_sc as plsc

_FIELD_DIMS = [100000] * 26
_EMBED_DIM = 32
_OFFSETS = np.array((0, *np.cumsum(_FIELD_DIMS)[:-1]), dtype=np.int32)

_B = 4096
_F = 26
_N = _B * _F
_NW = 32
_CHUNK = _N // _NW
_LANES = 16

_K = 8
_S = _CHUNK // _K

_OFF_REP = np.tile(_OFFSETS, _CHUNK // _F)

_mesh = plsc.VectorSubcoreMesh(core_axis_name="c", subcore_axis_name="s")


@functools.partial(
    pl.kernel,
    mesh=_mesh,
    out_type=jax.ShapeDtypeStruct((_N, _EMBED_DIM), jnp.float32),
    scratch_types=[
        pltpu.VMEM((_CHUNK,), jnp.int32),
        pltpu.VMEM((_CHUNK,), jnp.int32),
        pltpu.VMEM((2, _S, _EMBED_DIM), jnp.float32),
        pltpu.SemaphoreType.DMA((2,)),
        pltpu.SemaphoreType.DMA((2,)),
    ],
    compiler_params=pltpu.CompilerParams(use_tc_tiling_on_sc=False),
)
def _emb_lookup(x_hbm, off_hbm, table_hbm, out_hbm, idx_v, off_v, rows_v,
                gsem, psem):
    wid = lax.axis_index("s") * 2 + lax.axis_index("c")
    base = wid * _CHUNK
    pltpu.sync_copy(off_hbm, off_v)
    pltpu.sync_copy(x_hbm.at[pl.ds(base, _CHUNK)], idx_v)

    def prep(k):
        def body(i, carry):
            s = pl.ds(k * _S + i * _LANES, _LANES)
            idx_v[s] = idx_v[s] + off_v[s]
            return carry
        lax.fori_loop(0, _S // _LANES, body, 0)

    def start_gather(k):
        return pltpu.async_copy(
            table_hbm.at[idx_v.at[pl.ds(k * _S, _S)]],
            rows_v.at[k % 2], gsem.at[k % 2])

    def start_put(k):
        return pltpu.async_copy(
            rows_v.at[k % 2],
            out_hbm.at[pl.ds(base + k * _S, _S)], psem.at[k % 2])

    prep(0)
    g = [None] * _K
    p = [None] * _K
    g[0] = start_gather(0)
    prep(1)
    g[1] = start_gather(1)
    for k in range(_K):
        g[k].wait()
        p[k] = start_put(k)
        if k + 2 < _K:
            prep(k + 2)
            p[k].wait()
            g[k + 2] = start_gather(k + 2)
    p[_K - 2].wait()
    p[_K - 1].wait()


def kernel(x, table):
    xi = x.astype(jnp.int32).reshape(_N)
    out = _emb_lookup(xi, jnp.asarray(_OFF_REP), table)
    return out.reshape(_B, _F, _EMBED_DIM)
